# Initial kernel scaffold; baseline (speedup 1.0000x reference)
import functools

import jax
import jax.numpy as jnp
from jax import lax
from jax.experimental import pallas as pl
from jax.experimental.pallas import tpu as pltpu
from jax.experimental.pallas import tpu_sc as plsc

N, E, D, H, C, G, L = 10000, 320000, 128, 64, 6, 64, 3

NC, NS = 2, 16
NW = NC * NS
PER_W = E // NW
CH = 80
NCHUNK = PER_W // CH
ROWS_PER_TILE = N // NS


def _bn_relu(h, g, b):
    m = jnp.mean(h, axis=0, keepdims=True)
    v = jnp.mean((h - m) * (h - m), axis=0, keepdims=True)
    return jnp.maximum((h - m) * lax.rsqrt(v + 1e-5) * g + b, 0.0)


def _embed_body(x_ref, w_ref, b_ref, g_ref, bb_ref, o_ref):
    h = jnp.dot(x_ref[...], w_ref[...], preferred_element_type=jnp.float32)
    o_ref[...] = _bn_relu(h + b_ref[...], g_ref[...], bb_ref[...])


def _mlp_core(h, a0, a1, w1, b1, g1, bb1, w2, b2, g2, bb2):
    z = h + a0 + a1
    z = jnp.dot(z, w1, preferred_element_type=jnp.float32) + b1
    z = _bn_relu(z, g1, bb1)
    z = jnp.dot(z, w2, preferred_element_type=jnp.float32) + b2
    return _bn_relu(z, g2, bb2)


def _mlp_body(h_ref, a0_ref, a1_ref, w1_ref, b1_ref, g1_ref, bb1_ref,
              w2_ref, b2_ref, g2_ref, bb2_ref, o_ref):
    o_ref[...] = _mlp_core(h_ref[...], a0_ref[...], a1_ref[...],
                           w1_ref[...], b1_ref[...], g1_ref[...], bb1_ref[...],
                           w2_ref[...], b2_ref[...], g2_ref[...], bb2_ref[...])


def _final_body(h_ref, a0_ref, a1_ref, w1_ref, b1_ref, g1_ref, bb1_ref,
                w2_ref, b2_ref, g2_ref, bb2_ref, batch_ref,
                cw1_ref, cb1_ref, cw2_ref, cb2_ref, o_ref):
    h3 = _mlp_core(h_ref[...], a0_ref[...], a1_ref[...],
                   w1_ref[...], b1_ref[...], g1_ref[...], bb1_ref[...],
                   w2_ref[...], b2_ref[...], g2_ref[...], bb2_ref[...])
    gids = lax.broadcasted_iota(jnp.int32, (G, N), 0)
    oneh = (batch_ref[...] == gids).astype(jnp.float32)
    sums = jnp.dot(oneh, h3, preferred_element_type=jnp.float32)
    counts = jnp.maximum(jnp.sum(oneh, axis=1, keepdims=True), 1.0)
    pooled = sums / counts
    p = jnp.maximum(
        jnp.dot(pooled, cw1_ref[...], preferred_element_type=jnp.float32)
        + cb1_ref[...], 0.0)
    o_ref[...] = (jnp.dot(p, cw2_ref[...], preferred_element_type=jnp.float32)
                  + cb2_ref[...])


_embed_tc = pl.pallas_call(
    _embed_body, out_shape=jax.ShapeDtypeStruct((N, H), jnp.float32))

_mlp_tc = pl.pallas_call(
    _mlp_body, out_shape=jax.ShapeDtypeStruct((N, H), jnp.float32))

_final_tc = pl.pallas_call(
    _final_body, out_shape=jax.ShapeDtypeStruct((G, C), jnp.float32))


_sc_mesh = plsc.VectorSubcoreMesh(core_axis_name="c", subcore_axis_name="s")


@functools.partial(
    pl.kernel, mesh=_sc_mesh,
    out_type=jax.ShapeDtypeStruct((NC, N, H), jnp.float32),
    scratch_types=[
        pltpu.VMEM((CH,), jnp.int32),
        pltpu.VMEM((CH,), jnp.int32),
        pltpu.VMEM((CH, H), jnp.float32),
        pltpu.VMEM_SHARED((N, H), jnp.float32),
        pltpu.SemaphoreType.DMA,
    ],
)
def _agg_sc(h_hbm, src_hbm, dst_hbm, zro_hbm, out_hbm,
            sbuf, dbuf, rbuf, agg_sh, sem):
    cid = lax.axis_index("c")
    sid = lax.axis_index("s")
    wid = cid * NS + sid
    r0 = sid * ROWS_PER_TILE
    pltpu.sync_copy(zro_hbm.at[pl.ds(r0, ROWS_PER_TILE)],
                    agg_sh.at[pl.ds(r0, ROWS_PER_TILE)])
    plsc.subcore_barrier()

    base = wid * PER_W

    def body(j, carry):
        off = base + j * CH
        pltpu.sync_copy(src_hbm.at[pl.ds(off, CH)], sbuf)
        pltpu.sync_copy(dst_hbm.at[pl.ds(off, CH)], dbuf)
        pltpu.async_copy(h_hbm.at[sbuf], rbuf, sem).wait()
        pltpu.sync_copy(rbuf, agg_sh.at[dbuf], add=True)
        return carry

    lax.fori_loop(0, NCHUNK, body, 0)
    plsc.subcore_barrier()
    pltpu.sync_copy(agg_sh.at[pl.ds(r0, ROWS_PER_TILE)],
                    out_hbm.at[cid, pl.ds(r0, ROWS_PER_TILE)])


def kernel(x, edge_index, batch, embed_W, embed_b, embed_bn_g, embed_bn_b,
           fc1_W, fc1_b, mlp_bn_g, mlp_bn_b, fc2_W, fc2_b, out_bn_g, out_bn_b,
           cls_W1, cls_b1, cls_W2, cls_b2):
    src = edge_index[0]
    dst = edge_index[1]
    zeros = jnp.zeros((N, H), jnp.float32)
    row = lambda p: p.reshape(1, -1)

    h = _embed_tc(x, embed_W, row(embed_b), row(embed_bn_g), row(embed_bn_b))
    for i in range(L - 1):
        parts = _agg_sc(h, src, dst, zeros)
        h = _mlp_tc(h, parts[0], parts[1],
                    fc1_W[i], row(fc1_b[i]), row(mlp_bn_g[i]), row(mlp_bn_b[i]),
                    fc2_W[i], row(fc2_b[i]), row(out_bn_g[i]), row(out_bn_b[i]))
    parts = _agg_sc(h, src, dst, zeros)
    i = L - 1
    out = _final_tc(h, parts[0], parts[1],
                    fc1_W[i], row(fc1_b[i]), row(mlp_bn_g[i]), row(mlp_bn_b[i]),
                    fc2_W[i], row(fc2_b[i]), row(out_bn_g[i]), row(out_bn_b[i]),
                    batch.reshape(1, N), cls_W1, row(cls_b1), cls_W2, row(cls_b2))
    return out


# speedup vs baseline: 4.7983x; 4.7983x over previous
import functools

import jax
import jax.numpy as jnp
from jax import lax
from jax.experimental import pallas as pl
from jax.experimental.pallas import tpu as pltpu
from jax.experimental.pallas import tpu_sc as plsc

N, E, D, H, C, G, L = 10000, 320000, 128, 64, 6, 64, 3

NC, NS = 2, 16
NW = NC * NS
PER_W = E // NW
CH = 80
NCHUNK = PER_W // CH
NP = 10240
ROWS_PER_TILE = NP // NS


def _bn_relu(h, g, b):
    m = jnp.mean(h, axis=0, keepdims=True)
    v = jnp.mean((h - m) * (h - m), axis=0, keepdims=True)
    return jnp.maximum((h - m) * lax.rsqrt(v + 1e-5) * g + b, 0.0)


def _embed_body(x_ref, w_ref, b_ref, g_ref, bb_ref, o_ref):
    h = jnp.dot(x_ref[...], w_ref[...], preferred_element_type=jnp.float32)
    o_ref[...] = _bn_relu(h + b_ref[...], g_ref[...], bb_ref[...])


def _mlp_core(h, a0, a1, w1, b1, g1, bb1, w2, b2, g2, bb2):
    z = h + a0[:N] + a1[:N]
    z = jnp.dot(z, w1, preferred_element_type=jnp.float32) + b1
    z = _bn_relu(z, g1, bb1)
    z = jnp.dot(z, w2, preferred_element_type=jnp.float32) + b2
    return _bn_relu(z, g2, bb2)


def _mlp_body(h_ref, a0_ref, a1_ref, w1_ref, b1_ref, g1_ref, bb1_ref,
              w2_ref, b2_ref, g2_ref, bb2_ref, o_ref):
    o_ref[...] = _mlp_core(h_ref[...], a0_ref[...], a1_ref[...],
                           w1_ref[...], b1_ref[...], g1_ref[...], bb1_ref[...],
                           w2_ref[...], b2_ref[...], g2_ref[...], bb2_ref[...])


def _final_body(h_ref, a0_ref, a1_ref, w1_ref, b1_ref, g1_ref, bb1_ref,
                w2_ref, b2_ref, g2_ref, bb2_ref, batch_ref,
                cw1_ref, cb1_ref, cw2_ref, cb2_ref, o_ref):
    h3 = _mlp_core(h_ref[...], a0_ref[...], a1_ref[...],
                   w1_ref[...], b1_ref[...], g1_ref[...], bb1_ref[...],
                   w2_ref[...], b2_ref[...], g2_ref[...], bb2_ref[...])
    gids = lax.broadcasted_iota(jnp.int32, (G, N), 0)
    oneh = (batch_ref[...] == gids).astype(jnp.float32)
    sums = jnp.dot(oneh, h3, preferred_element_type=jnp.float32)
    counts = jnp.maximum(jnp.sum(oneh, axis=1, keepdims=True), 1.0)
    pooled = sums / counts
    p = jnp.maximum(
        jnp.dot(pooled, cw1_ref[...], preferred_element_type=jnp.float32)
        + cb1_ref[...], 0.0)
    o_ref[...] = (jnp.dot(p, cw2_ref[...], preferred_element_type=jnp.float32)
                  + cb2_ref[...])


_embed_tc = pl.pallas_call(
    _embed_body, out_shape=jax.ShapeDtypeStruct((N, H), jnp.float32))

_mlp_tc = pl.pallas_call(
    _mlp_body, out_shape=jax.ShapeDtypeStruct((N, H), jnp.float32))

_final_tc = pl.pallas_call(
    _final_body, out_shape=jax.ShapeDtypeStruct((G, C), jnp.float32))


@functools.lru_cache(maxsize=None)
def _make_agg_sc():
    mesh = plsc.VectorSubcoreMesh(core_axis_name="c", subcore_axis_name="s",
                                  num_cores=NC, num_subcores=NS)

    @functools.partial(
        pl.kernel, mesh=mesh,
        out_type=jax.ShapeDtypeStruct((NC, NP, H), jnp.float32),
        scratch_types=[
            pltpu.VMEM((CH,), jnp.int32),
            pltpu.VMEM((CH,), jnp.int32),
            pltpu.VMEM((CH, H), jnp.float32),
            pltpu.VMEM_SHARED((NP, H), jnp.float32),
            pltpu.SemaphoreType.DMA,
        ],
        compiler_params=pltpu.CompilerParams(use_tc_tiling_on_sc=False),
    )
    def _agg_sc(h_hbm, src_hbm, dst_hbm, zro_hbm, out_hbm,
                sbuf, dbuf, rbuf, agg_sh, sem):
        cid = lax.axis_index("c")
        sid = lax.axis_index("s")
        wid = cid * NS + sid
        r0 = sid * ROWS_PER_TILE
        pltpu.sync_copy(zro_hbm.at[pl.ds(r0, ROWS_PER_TILE)],
                        agg_sh.at[pl.ds(r0, ROWS_PER_TILE)])
        plsc.subcore_barrier()

        base = wid * PER_W

        def body(j, carry):
            off = base + j * CH
            pltpu.sync_copy(src_hbm.at[pl.ds(off, CH)], sbuf)
            pltpu.sync_copy(dst_hbm.at[pl.ds(off, CH)], dbuf)
            pltpu.async_copy(h_hbm.at[sbuf], rbuf, sem).wait()
            pltpu.sync_copy(rbuf, agg_sh.at[dbuf], add=True)
            return carry

        lax.fori_loop(0, NCHUNK, body, 0)
        plsc.subcore_barrier()
        pltpu.sync_copy(agg_sh.at[pl.ds(r0, ROWS_PER_TILE)],
                        out_hbm.at[cid, pl.ds(r0, ROWS_PER_TILE)])

    return _agg_sc


def kernel(x, edge_index, batch, embed_W, embed_b, embed_bn_g, embed_bn_b,
           fc1_W, fc1_b, mlp_bn_g, mlp_bn_b, fc2_W, fc2_b, out_bn_g, out_bn_b,
           cls_W1, cls_b1, cls_W2, cls_b2):
    src = edge_index[0]
    dst = edge_index[1]
    zeros = jnp.zeros((NP, H), jnp.float32)
    row = lambda p: p.reshape(1, -1)
    _agg_sc = _make_agg_sc()

    h = _embed_tc(x, embed_W, row(embed_b), row(embed_bn_g), row(embed_bn_b))
    for i in range(L - 1):
        parts = _agg_sc(h, src, dst, zeros)
        h = _mlp_tc(h, parts[0], parts[1],
                    fc1_W[i], row(fc1_b[i]), row(mlp_bn_g[i]), row(mlp_bn_b[i]),
                    fc2_W[i], row(fc2_b[i]), row(out_bn_g[i]), row(out_bn_b[i]))
    parts = _agg_sc(h, src, dst, zeros)
    i = L - 1
    out = _final_tc(h, parts[0], parts[1],
                    fc1_W[i], row(fc1_b[i]), row(mlp_bn_g[i]), row(mlp_bn_b[i]),
                    fc2_W[i], row(fc2_b[i]), row(out_bn_g[i]), row(out_bn_b[i]),
                    batch.reshape(1, N), cls_W1, row(cls_b1), cls_W2, row(cls_b2))
    return out


# speedup vs baseline: 10.8926x; 2.2701x over previous
import functools

import jax
import jax.numpy as jnp
from jax import lax
from jax.experimental import pallas as pl
from jax.experimental.pallas import tpu as pltpu
from jax.experimental.pallas import tpu_sc as plsc

N, E, D, H, C, G, L = 10000, 320000, 128, 64, 6, 64, 3

NC, NS = 2, 16
NW = NC * NS
PER_W = E // NW
CH = 80
NCHUNK = PER_W // CH
NP = 10240
ROWS_PER_TILE = NP // NS


def _bn_relu(h, g, b):
    m = jnp.mean(h, axis=0, keepdims=True)
    v = jnp.mean((h - m) * (h - m), axis=0, keepdims=True)
    return jnp.maximum((h - m) * lax.rsqrt(v + 1e-5) * g + b, 0.0)


def _embed_body(x_ref, w_ref, b_ref, g_ref, bb_ref, o_ref):
    h = jnp.dot(x_ref[...], w_ref[...], preferred_element_type=jnp.float32)
    o_ref[...] = _bn_relu(h + b_ref[...], g_ref[...], bb_ref[...])


def _mlp_core(h, a0, a1, w1, b1, g1, bb1, w2, b2, g2, bb2):
    z = h + a0[:N] + a1[:N]
    z = jnp.dot(z, w1, preferred_element_type=jnp.float32) + b1
    z = _bn_relu(z, g1, bb1)
    z = jnp.dot(z, w2, preferred_element_type=jnp.float32) + b2
    return _bn_relu(z, g2, bb2)


def _mlp_body(h_ref, a0_ref, a1_ref, w1_ref, b1_ref, g1_ref, bb1_ref,
              w2_ref, b2_ref, g2_ref, bb2_ref, o_ref):
    o_ref[...] = _mlp_core(h_ref[...], a0_ref[...], a1_ref[...],
                           w1_ref[...], b1_ref[...], g1_ref[...], bb1_ref[...],
                           w2_ref[...], b2_ref[...], g2_ref[...], bb2_ref[...])


def _final_body(h_ref, a0_ref, a1_ref, w1_ref, b1_ref, g1_ref, bb1_ref,
                w2_ref, b2_ref, g2_ref, bb2_ref, batch_ref,
                cw1_ref, cb1_ref, cw2_ref, cb2_ref, o_ref):
    h3 = _mlp_core(h_ref[...], a0_ref[...], a1_ref[...],
                   w1_ref[...], b1_ref[...], g1_ref[...], bb1_ref[...],
                   w2_ref[...], b2_ref[...], g2_ref[...], bb2_ref[...])
    gids = lax.broadcasted_iota(jnp.int32, (G, N), 0)
    oneh = (batch_ref[...] == gids).astype(jnp.float32)
    sums = jnp.dot(oneh, h3, preferred_element_type=jnp.float32)
    counts = jnp.maximum(jnp.sum(oneh, axis=1, keepdims=True), 1.0)
    pooled = sums / counts
    p = jnp.maximum(
        jnp.dot(pooled, cw1_ref[...], preferred_element_type=jnp.float32)
        + cb1_ref[...], 0.0)
    o_ref[...] = (jnp.dot(p, cw2_ref[...], preferred_element_type=jnp.float32)
                  + cb2_ref[...])


_embed_tc = pl.pallas_call(
    _embed_body, out_shape=jax.ShapeDtypeStruct((N, H), jnp.float32))

_mlp_tc = pl.pallas_call(
    _mlp_body, out_shape=jax.ShapeDtypeStruct((N, H), jnp.float32))

_final_tc = pl.pallas_call(
    _final_body, out_shape=jax.ShapeDtypeStruct((G, C), jnp.float32))


@functools.lru_cache(maxsize=None)
def _make_agg_sc():
    mesh = plsc.VectorSubcoreMesh(core_axis_name="c", subcore_axis_name="s",
                                  num_cores=NC, num_subcores=NS)

    @functools.partial(
        pl.kernel, mesh=mesh,
        out_type=jax.ShapeDtypeStruct((NC, NP, H), jnp.float32),
        scratch_types=[
            pltpu.VMEM((NCHUNK, CH), jnp.int32),
            pltpu.VMEM((NCHUNK, CH), jnp.int32),
            pltpu.VMEM((CH, H), jnp.float32),
            pltpu.VMEM((CH, H), jnp.float32),
            pltpu.VMEM_SHARED((NP, H), jnp.float32),
            pltpu.SemaphoreType.DMA,
            pltpu.SemaphoreType.DMA,
        ],
        compiler_params=pltpu.CompilerParams(use_tc_tiling_on_sc=False),
    )
    def _agg_sc(h_hbm, src_hbm, dst_hbm, zro_hbm, out_hbm,
                sbuf, dbuf, rb0, rb1, agg_sh, gs0, gs1):
        cid = lax.axis_index("c")
        sid = lax.axis_index("s")
        wid = cid * NS + sid
        r0 = sid * ROWS_PER_TILE
        pltpu.sync_copy(zro_hbm.at[pl.ds(r0, ROWS_PER_TILE)],
                        agg_sh.at[pl.ds(r0, ROWS_PER_TILE)])
        crow = wid * NCHUNK
        pltpu.sync_copy(src_hbm.at[pl.ds(crow, NCHUNK)], sbuf)
        pltpu.sync_copy(dst_hbm.at[pl.ds(crow, NCHUNK)], dbuf)
        plsc.subcore_barrier()

        def drain(rb, sem):
            pltpu.make_async_copy(h_hbm.at[pl.ds(0, CH)], rb, sem).wait()

        pltpu.async_copy(h_hbm.at[sbuf.at[0]], rb0, gs0)
        pltpu.async_copy(h_hbm.at[sbuf.at[1]], rb1, gs1)

        def body(i, carry):
            j0 = 2 * i
            drain(rb0, gs0)
            pltpu.sync_copy(rb0, agg_sh.at[dbuf.at[j0]], add=True)

            @pl.when(j0 + 2 < NCHUNK)
            def _():
                pltpu.async_copy(h_hbm.at[sbuf.at[j0 + 2]], rb0, gs0)

            j1 = j0 + 1

            @pl.when(j1 < NCHUNK)
            def _():
                drain(rb1, gs1)
                pltpu.sync_copy(rb1, agg_sh.at[dbuf.at[j1]], add=True)

                @pl.when(j1 + 2 < NCHUNK)
                def _():
                    pltpu.async_copy(h_hbm.at[sbuf.at[j1 + 2]], rb1, gs1)

            return carry

        lax.fori_loop(0, (NCHUNK + 1) // 2, body, 0)
        plsc.subcore_barrier()
        pltpu.sync_copy(agg_sh.at[pl.ds(r0, ROWS_PER_TILE)],
                        out_hbm.at[cid, pl.ds(r0, ROWS_PER_TILE)])

    return _agg_sc


def kernel(x, edge_index, batch, embed_W, embed_b, embed_bn_g, embed_bn_b,
           fc1_W, fc1_b, mlp_bn_g, mlp_bn_b, fc2_W, fc2_b, out_bn_g, out_bn_b,
           cls_W1, cls_b1, cls_W2, cls_b2):
    src = edge_index[0].reshape(E // CH, CH)
    dst = edge_index[1].reshape(E // CH, CH)
    zeros = jnp.zeros((NP, H), jnp.float32)
    row = lambda p: p.reshape(1, -1)
    _agg_sc = _make_agg_sc()

    h = _embed_tc(x, embed_W, row(embed_b), row(embed_bn_g), row(embed_bn_b))
    for i in range(L - 1):
        parts = _agg_sc(h, src, dst, zeros)
        h = _mlp_tc(h, parts[0], parts[1],
                    fc1_W[i], row(fc1_b[i]), row(mlp_bn_g[i]), row(mlp_bn_b[i]),
                    fc2_W[i], row(fc2_b[i]), row(out_bn_g[i]), row(out_bn_b[i]))
    parts = _agg_sc(h, src, dst, zeros)
    i = L - 1
    out = _final_tc(h, parts[0], parts[1],
                    fc1_W[i], row(fc1_b[i]), row(mlp_bn_g[i]), row(mlp_bn_b[i]),
                    fc2_W[i], row(fc2_b[i]), row(out_bn_g[i]), row(out_bn_b[i]),
                    batch.reshape(1, N), cls_W1, row(cls_b1), cls_W2, row(cls_b2))
    return out
